# baseline (speedup 1.0000x reference)
import functools

import jax
import jax.numpy as jnp
from jax import lax
from jax.experimental import pallas as pl
from jax.experimental.pallas import tpu as pltpu
from jax.experimental.pallas import tpu_sc as plsc

_NC, _NS = 2, 16
_NW = _NC * _NS
_CH = 128


def _sc_gather(ids3, itax, word_emb, pos_z):
    nw, n_chunks, _ = ids3.shape
    _, n_pchunks, align, _ = itax.shape
    bl = nw * n_chunks * _CH
    bp = nw * n_pchunks * _CH
    _, h = word_emb.shape
    rows_w = bl // _NW
    prow_w = bp // _NW

    mesh = plsc.VectorSubcoreMesh(
        core_axis_name="c", subcore_axis_name="s",
        num_cores=_NC, num_subcores=_NS)

    @functools.partial(
        pl.kernel,
        out_type=(jax.ShapeDtypeStruct((bl, h), jnp.float32),
                  jax.ShapeDtypeStruct((bp, h), jnp.float32)),
        mesh=mesh,
        scratch_types=[
            pltpu.VMEM((n_chunks, _CH), jnp.int32),
            pltpu.VMEM((n_pchunks, align, _CH), jnp.int32),
            pltpu.VMEM((_CH, h), jnp.float32),
            pltpu.VMEM((_CH, h), jnp.float32),
            pltpu.VMEM((_CH, h), jnp.float32),
            pltpu.VMEM((_CH, h), jnp.float32),
            pltpu.SemaphoreType.DMA,
            pltpu.SemaphoreType.DMA,
            pltpu.SemaphoreType.DMA,
            pltpu.SemaphoreType.DMA,
            pltpu.SemaphoreType.DMA,
            pltpu.SemaphoreType.DMA,
        ],
    )
    def k(ids_hbm, ita_hbm, word_hbm, posz_hbm, zero_hbm, g_hbm, p_hbm,
          idx_v, pidx_v, bufa, bufb, acca, accb,
          sema, semb, psema, psemb, zsema, zsemb):
        wid = lax.axis_index("s") * _NC + lax.axis_index("c")
        gbase = wid * rows_w
        pbase = wid * prow_w
        pltpu.sync_copy(ids_hbm.at[wid], idx_v)
        pltpu.sync_copy(ita_hbm.at[wid], pidx_v)

        pltpu.async_copy(word_hbm.at[idx_v.at[0]], bufa, sema)

        def tbody(g2, c):
            j = 2 * g2
            pltpu.async_copy(word_hbm.at[idx_v.at[j + 1]], bufb, semb)
            pltpu.make_async_copy(word_hbm.at[idx_v.at[j]], bufa, sema).wait()
            pltpu.sync_copy(bufa, g_hbm.at[pl.ds(gbase + j * _CH, _CH)])

            @pl.when(j + 2 < n_chunks)
            def _():
                pltpu.async_copy(word_hbm.at[idx_v.at[j + 2]], bufa, sema)

            pltpu.make_async_copy(
                word_hbm.at[idx_v.at[j + 1]], bufb, semb).wait()
            pltpu.sync_copy(bufb, g_hbm.at[pl.ds(gbase + (j + 1) * _CH, _CH)])
            return c

        lax.fori_loop(0, n_chunks // 2, tbody, 0)

        pltpu.async_copy(zero_hbm, acca, zsema)
        pltpu.async_copy(zero_hbm, accb, zsemb)

        def fire(j, accx, psx, zsx):
            pltpu.make_async_copy(zero_hbm, accx, zsx).wait()
            for a in range(align):
                pltpu.async_copy(
                    posz_hbm.at[pidx_v.at[j, a]], accx, psx, add=True)

        def drain_write(j, accx, psx):
            for a in range(align):
                pltpu.make_async_copy(zero_hbm, accx, psx).wait()
            pltpu.sync_copy(accx, p_hbm.at[pl.ds(pbase + j * _CH, _CH)])

        def pbody(g2, c):
            j = 2 * g2
            fire(j, acca, psema, zsema)
            fire(j + 1, accb, psemb, zsemb)
            drain_write(j, acca, psema)
            pltpu.async_copy(zero_hbm, acca, zsema)
            drain_write(j + 1, accb, psemb)
            pltpu.async_copy(zero_hbm, accb, zsemb)
            return c

        lax.fori_loop(0, n_pchunks // 2, pbody, 0)
        if n_pchunks % 2:
            j = n_pchunks - 1
            fire(j, acca, psema, zsema)
            drain_write(j, acca, psema)

    return k(ids3, itax, word_emb, pos_z, jnp.zeros((_CH, h), jnp.float32))


def _tc_body(g_ref, vis_ref, ttt_ref, vtt_ref, rms_ref, p_ref, pos_ref,
             aux_ref, w_ref, out_ref):
    l, grp, h = g_ref.shape
    v = vis_ref.shape[0]
    d = vis_ref.shape[2]

    type2 = aux_ref[0:2, :]
    pev0 = aux_ref[2:3, :]
    pb = aux_ref[3:4, :]
    gam = aux_ref[4:5, :][:, None, :]
    bet = aux_ref[5:6, :][:, None, :]

    def ln3(x):
        m = jnp.mean(x, axis=-1, keepdims=True)
        xc = x - m
        var = jnp.mean(xc * xc, axis=-1, keepdims=True)
        return xc * lax.rsqrt(var + 1e-12) * gam + bet

    dn = (((0,), (0,)), ((), ()))
    ttype = lax.dot_general(ttt_ref[0], type2, dn,
                            preferred_element_type=jnp.float32)
    text = g_ref[...] + pos_ref[...][:, None, :] + ttype.reshape(l, grp, h)

    vf = jnp.dot(vis_ref[...].reshape(v * grp, d), w_ref[...],
                 preferred_element_type=jnp.float32)
    vtype = lax.dot_general(vtt_ref[0], type2, dn,
                            preferred_element_type=jnp.float32)
    rowi = lax.broadcasted_iota(jnp.int32, (v * grp, v * grp), 0)
    colj = lax.broadcasted_iota(jnp.int32, (v * grp, v * grp), 1)
    sel = jnp.where(colj == rowi,
                    jnp.broadcast_to(rms_ref[0], (v * grp, v * grp)), 0.0)
    pev = jnp.dot(sel, p_ref[...].reshape(v * grp, h),
                  preferred_element_type=jnp.float32)
    vis = vf + pev + vtype + (pb + pev0)

    out_ref[0:l] = ln3(text)
    out_ref[l:l + v] = ln3(vis.reshape(v, grp, h))


def kernel(input_ids, token_type_ids, visual_feats, visual_feature_type_ids,
           image_text_alignment, word_emb, pos_emb, type_emb, pos_emb_visual,
           proj_W, proj_b, ln_gamma, ln_beta):
    b, l = input_ids.shape
    v = visual_feats.shape[1]
    h = word_emb.shape[1]
    align = image_text_alignment.shape[2]

    ids3 = (input_ids.astype(jnp.int32).T
            .reshape(_NW, b * l // (_NW * _CH), _CH))
    itax = (image_text_alignment.astype(jnp.int32).transpose(1, 0, 2)
            .reshape(_NW, b * v // (_NW * _CH), _CH, align)
            .transpose(0, 1, 3, 2))
    pos_z = pos_emb.at[0].set(0.0)

    g, p = _sc_gather(ids3, itax, word_emb, pos_z)
    g = g.reshape(l, b, h)
    p = p.reshape(v, b, h)
    vis_t = visual_feats.astype(jnp.bfloat16).transpose(1, 0, 2)

    grp = 8
    ng = b // grp
    two = jnp.arange(2, dtype=token_type_ids.dtype)
    ttt = (token_type_ids.T.reshape(l, ng, grp)[None]
           == two[:, None, None, None]).astype(jnp.float32)
    ttt = ttt.transpose(2, 0, 1, 3).reshape(ng, 2, l * grp)
    vtt = (visual_feature_type_ids.T.reshape(v, ng, grp)[None]
           == two[:, None, None, None]).astype(jnp.float32)
    vtt = vtt.transpose(2, 0, 1, 3).reshape(ng, 2, v * grp)
    msum = (image_text_alignment != 0).sum(axis=2).astype(jnp.float32)
    rms = (1.0 / jnp.maximum(msum, 1.0)).T.reshape(v, ng, grp)
    rms = rms.transpose(1, 0, 2).reshape(ng, 1, v * grp)

    aux = jnp.zeros((8, h), jnp.float32)
    aux = aux.at[0].set(type_emb[0]).at[1].set(type_emb[1])
    aux = aux.at[2].set(pos_emb_visual[0]).at[3].set(proj_b)
    aux = aux.at[4].set(ln_gamma).at[5].set(ln_beta)

    pos_l = pos_emb[:l]
    d = visual_feats.shape[2]

    out = pl.pallas_call(
        _tc_body,
        grid=(ng,),
        in_specs=[
            pl.BlockSpec((l, grp, h), lambda i: (0, i, 0)),
            pl.BlockSpec((v, grp, d), lambda i: (0, i, 0)),
            pl.BlockSpec((1, 2, grp * l), lambda i: (i, 0, 0)),
            pl.BlockSpec((1, 2, grp * v), lambda i: (i, 0, 0)),
            pl.BlockSpec((1, 1, grp * v), lambda i: (i, 0, 0)),
            pl.BlockSpec((v, grp, h), lambda i: (0, i, 0)),
            pl.BlockSpec((l, h), lambda i: (0, 0)),
            pl.BlockSpec((8, h), lambda i: (0, 0)),
            pl.BlockSpec((d, h), lambda i: (0, 0)),
        ],
        out_specs=pl.BlockSpec((l + v, grp, h), lambda i: (0, i, 0)),
        out_shape=jax.ShapeDtypeStruct((l + v, b, h), jnp.float32),
        compiler_params=pltpu.CompilerParams(
            dimension_semantics=("arbitrary",)),
    )(g, vis_t, ttt, vtt, rms, p, pos_l, aux,
      proj_W.astype(jnp.bfloat16))
    return jnp.transpose(out, (1, 0, 2))


# speedup vs baseline: 1.1349x; 1.1349x over previous
import functools

import jax
import jax.numpy as jnp
from jax import lax
from jax.experimental import pallas as pl
from jax.experimental.pallas import tpu as pltpu
from jax.experimental.pallas import tpu_sc as plsc

_NC, _NS = 2, 16
_NW = _NC * _NS
_CH = 128


def _sc_gather(ids3, itax, word_emb, pos_z):
    nw, n_chunks, _ = ids3.shape
    _, n_pchunks, align, _ = itax.shape
    bl = nw * n_chunks * _CH
    bp = nw * n_pchunks * _CH
    _, h = word_emb.shape
    rows_w = bl // _NW
    prow_w = bp // _NW

    mesh = plsc.VectorSubcoreMesh(
        core_axis_name="c", subcore_axis_name="s",
        num_cores=_NC, num_subcores=_NS)

    @functools.partial(
        pl.kernel,
        out_type=(jax.ShapeDtypeStruct((bl, h), jnp.float32),
                  jax.ShapeDtypeStruct((bp, h), jnp.float32)),
        mesh=mesh,
        scratch_types=[
            pltpu.VMEM((n_chunks, _CH), jnp.int32),
            pltpu.VMEM((n_pchunks, align, _CH), jnp.int32),
            pltpu.VMEM((_CH, h), jnp.float32),
            pltpu.VMEM((_CH, h), jnp.float32),
            pltpu.VMEM((_CH, h), jnp.float32),
            pltpu.VMEM((_CH, h), jnp.float32),
            pltpu.SemaphoreType.DMA,
            pltpu.SemaphoreType.DMA,
            pltpu.SemaphoreType.DMA,
            pltpu.SemaphoreType.DMA,
            pltpu.SemaphoreType.DMA,
            pltpu.SemaphoreType.DMA,
        ],
    )
    def k(ids_hbm, ita_hbm, word_hbm, posz_hbm, zero_hbm, g_hbm, p_hbm,
          idx_v, pidx_v, bufa, bufb, acca, accb,
          sema, semb, psema, psemb, zsema, zsemb):
        wid = lax.axis_index("s") * _NC + lax.axis_index("c")
        gbase = wid * rows_w
        pbase = wid * prow_w
        pltpu.sync_copy(ids_hbm.at[wid], idx_v)
        pltpu.sync_copy(ita_hbm.at[wid], pidx_v)

        pltpu.async_copy(word_hbm.at[idx_v.at[0]], bufa, sema)

        def tbody(g2, c):
            j = 2 * g2
            pltpu.async_copy(word_hbm.at[idx_v.at[j + 1]], bufb, semb)
            pltpu.make_async_copy(word_hbm.at[idx_v.at[j]], bufa, sema).wait()
            pltpu.sync_copy(bufa, g_hbm.at[pl.ds(gbase + j * _CH, _CH)])

            @pl.when(j + 2 < n_chunks)
            def _():
                pltpu.async_copy(word_hbm.at[idx_v.at[j + 2]], bufa, sema)

            pltpu.make_async_copy(
                word_hbm.at[idx_v.at[j + 1]], bufb, semb).wait()
            pltpu.sync_copy(bufb, g_hbm.at[pl.ds(gbase + (j + 1) * _CH, _CH)])
            return c

        lax.fori_loop(0, n_chunks // 2, tbody, 0)

        pltpu.async_copy(zero_hbm, acca, zsema)
        pltpu.async_copy(zero_hbm, accb, zsemb)

        def fire(j, accx, psx, zsx):
            pltpu.make_async_copy(zero_hbm, accx, zsx).wait()
            for a in range(align):
                pltpu.async_copy(
                    posz_hbm.at[pidx_v.at[j, a]], accx, psx, add=True)

        def drain_write(j, accx, psx):
            for a in range(align):
                pltpu.make_async_copy(zero_hbm, accx, psx).wait()
            pltpu.sync_copy(accx, p_hbm.at[pl.ds(pbase + j * _CH, _CH)])

        def pbody(g2, c):
            j = 2 * g2
            fire(j, acca, psema, zsema)
            fire(j + 1, accb, psemb, zsemb)
            drain_write(j, acca, psema)
            pltpu.async_copy(zero_hbm, acca, zsema)
            drain_write(j + 1, accb, psemb)
            pltpu.async_copy(zero_hbm, accb, zsemb)
            return c

        lax.fori_loop(0, n_pchunks // 2, pbody, 0)
        if n_pchunks % 2:
            j = n_pchunks - 1
            fire(j, acca, psema, zsema)
            drain_write(j, acca, psema)

    return k(ids3, itax, word_emb, pos_z, jnp.zeros((_CH, h), jnp.float32))


def _tc_body(g_ref, vis_ref, ttt_ref, vtt_ref, rms_ref, p_ref, pos_ref,
             aux_ref, w_ref, out_ref):
    l, grp, h = g_ref.shape
    v = vis_ref.shape[0]
    d = vis_ref.shape[2]

    type2 = aux_ref[0:2, :]
    pev0 = aux_ref[2:3, :]
    pb = aux_ref[3:4, :]
    gam = aux_ref[4:5, :][:, None, :]
    bet = aux_ref[5:6, :][:, None, :]

    def ln3(x):
        m = jnp.mean(x, axis=-1, keepdims=True)
        xc = x - m
        var = jnp.mean(xc * xc, axis=-1, keepdims=True)
        return xc * lax.rsqrt(var + 1e-12) * gam + bet

    dn = (((0,), (0,)), ((), ()))
    ttype = lax.dot_general(ttt_ref[0], type2, dn,
                            preferred_element_type=jnp.float32)
    text = g_ref[...] + pos_ref[...][:, None, :] + ttype.reshape(l, grp, h)

    vf = jnp.dot(vis_ref[...].reshape(v * grp, d).astype(jnp.bfloat16),
                 w_ref[...], preferred_element_type=jnp.float32)
    vtype = lax.dot_general(vtt_ref[0], type2, dn,
                            preferred_element_type=jnp.float32)
    rowi = lax.broadcasted_iota(jnp.int32, (v * grp, v * grp), 0)
    colj = lax.broadcasted_iota(jnp.int32, (v * grp, v * grp), 1)
    sel = jnp.where(colj == rowi,
                    jnp.broadcast_to(rms_ref[0], (v * grp, v * grp)), 0.0)
    pev = jnp.dot(sel, p_ref[...].reshape(v * grp, h),
                  preferred_element_type=jnp.float32)
    vis = vf + pev + vtype + (pb + pev0)

    out_ref[0:l] = ln3(text)
    out_ref[l:l + v] = ln3(vis.reshape(v, grp, h))


def kernel(input_ids, token_type_ids, visual_feats, visual_feature_type_ids,
           image_text_alignment, word_emb, pos_emb, type_emb, pos_emb_visual,
           proj_W, proj_b, ln_gamma, ln_beta):
    b, l = input_ids.shape
    v = visual_feats.shape[1]
    h = word_emb.shape[1]
    align = image_text_alignment.shape[2]

    ids3 = (input_ids.astype(jnp.int32).T
            .reshape(_NW, b * l // (_NW * _CH), _CH))
    itax = (image_text_alignment.astype(jnp.int32).transpose(1, 0, 2)
            .reshape(_NW, b * v // (_NW * _CH), _CH, align)
            .transpose(0, 1, 3, 2))
    pos_z = pos_emb.at[0].set(0.0)

    g, p = _sc_gather(ids3, itax, word_emb, pos_z)
    g = g.reshape(l, b, h)
    p = p.reshape(v, b, h)
    vis_t = visual_feats.transpose(1, 0, 2)

    grp = 8
    ng = b // grp
    two = jnp.arange(2, dtype=token_type_ids.dtype)
    ttt = (token_type_ids.T.reshape(l, ng, grp)[None]
           == two[:, None, None, None]).astype(jnp.float32)
    ttt = ttt.transpose(2, 0, 1, 3).reshape(ng, 2, l * grp)
    vtt = (visual_feature_type_ids.T.reshape(v, ng, grp)[None]
           == two[:, None, None, None]).astype(jnp.float32)
    vtt = vtt.transpose(2, 0, 1, 3).reshape(ng, 2, v * grp)
    msum = (image_text_alignment != 0).sum(axis=2).astype(jnp.float32)
    rms = (1.0 / jnp.maximum(msum, 1.0)).T.reshape(v, ng, grp)
    rms = rms.transpose(1, 0, 2).reshape(ng, 1, v * grp)

    aux = jnp.zeros((8, h), jnp.float32)
    aux = aux.at[0].set(type_emb[0]).at[1].set(type_emb[1])
    aux = aux.at[2].set(pos_emb_visual[0]).at[3].set(proj_b)
    aux = aux.at[4].set(ln_gamma).at[5].set(ln_beta)

    pos_l = pos_emb[:l]
    d = visual_feats.shape[2]

    out = pl.pallas_call(
        _tc_body,
        grid=(ng,),
        in_specs=[
            pl.BlockSpec((l, grp, h), lambda i: (0, i, 0)),
            pl.BlockSpec((v, grp, d), lambda i: (0, i, 0)),
            pl.BlockSpec((1, 2, grp * l), lambda i: (i, 0, 0)),
            pl.BlockSpec((1, 2, grp * v), lambda i: (i, 0, 0)),
            pl.BlockSpec((1, 1, grp * v), lambda i: (i, 0, 0)),
            pl.BlockSpec((v, grp, h), lambda i: (0, i, 0)),
            pl.BlockSpec((l, h), lambda i: (0, 0)),
            pl.BlockSpec((8, h), lambda i: (0, 0)),
            pl.BlockSpec((d, h), lambda i: (0, 0)),
        ],
        out_specs=pl.BlockSpec((l + v, grp, h), lambda i: (0, i, 0)),
        out_shape=jax.ShapeDtypeStruct((l + v, b, h), jnp.float32),
        compiler_params=pltpu.CompilerParams(
            dimension_semantics=("arbitrary",)),
    )(g, vis_t, ttt, vtt, rms, p, pos_l, aux,
      proj_W.astype(jnp.bfloat16))
    return jnp.transpose(out, (1, 0, 2))


# speedup vs baseline: 1.2591x; 1.1095x over previous
import functools

import jax
import jax.numpy as jnp
from jax import lax
from jax.experimental import pallas as pl
from jax.experimental.pallas import tpu as pltpu
from jax.experimental.pallas import tpu_sc as plsc

_NC, _NS = 2, 16
_NW = _NC * _NS
_CH = 128


def _sc_gather(ids3, itax, word_emb, pos_z):
    nw, n_chunks, _ = ids3.shape
    _, n_pchunks, align, _ = itax.shape
    bl = nw * n_chunks * _CH
    bp = nw * n_pchunks * _CH
    _, h = word_emb.shape
    rows_w = bl // _NW
    prow_w = bp // _NW

    mesh = plsc.VectorSubcoreMesh(
        core_axis_name="c", subcore_axis_name="s",
        num_cores=_NC, num_subcores=_NS)

    @functools.partial(
        pl.kernel,
        out_type=(jax.ShapeDtypeStruct((bl, h), jnp.float32),
                  jax.ShapeDtypeStruct((bp, h), jnp.float32)),
        mesh=mesh,
        scratch_types=[
            pltpu.VMEM((n_chunks, _CH), jnp.int32),
            pltpu.VMEM((n_pchunks, align, _CH), jnp.int32),
            pltpu.VMEM((_CH, h), jnp.float32),
            pltpu.VMEM((_CH, h), jnp.float32),
            pltpu.VMEM((_CH, h), jnp.float32),
            pltpu.VMEM((_CH, h), jnp.float32),
            pltpu.SemaphoreType.DMA,
            pltpu.SemaphoreType.DMA,
            pltpu.SemaphoreType.DMA,
            pltpu.SemaphoreType.DMA,
            pltpu.SemaphoreType.DMA,
            pltpu.SemaphoreType.DMA,
        ],
    )
    def k(ids_hbm, ita_hbm, word_hbm, posz_hbm, zero_hbm, g_hbm, p_hbm,
          idx_v, pidx_v, bufa, bufb, acca, accb,
          sema, semb, psema, psemb, zsema, zsemb):
        wid = lax.axis_index("s") * _NC + lax.axis_index("c")
        gbase = wid * rows_w
        pbase = wid * prow_w
        pltpu.sync_copy(ids_hbm.at[wid], idx_v)
        pltpu.sync_copy(ita_hbm.at[wid], pidx_v)

        pltpu.async_copy(word_hbm.at[idx_v.at[0]], bufa, sema)

        def tbody(g2, c):
            j = 2 * g2
            pltpu.async_copy(word_hbm.at[idx_v.at[j + 1]], bufb, semb)
            pltpu.make_async_copy(word_hbm.at[idx_v.at[j]], bufa, sema).wait()
            pltpu.sync_copy(bufa, g_hbm.at[pl.ds(gbase + j * _CH, _CH)])

            @pl.when(j + 2 < n_chunks)
            def _():
                pltpu.async_copy(word_hbm.at[idx_v.at[j + 2]], bufa, sema)

            pltpu.make_async_copy(
                word_hbm.at[idx_v.at[j + 1]], bufb, semb).wait()
            pltpu.sync_copy(bufb, g_hbm.at[pl.ds(gbase + (j + 1) * _CH, _CH)])
            return c

        lax.fori_loop(0, n_chunks // 2, tbody, 0)

        pltpu.async_copy(zero_hbm, acca, zsema)
        pltpu.async_copy(zero_hbm, accb, zsemb)

        def fire(j, accx, psx, zsx):
            pltpu.make_async_copy(zero_hbm, accx, zsx).wait()
            for a in range(align):
                pltpu.async_copy(
                    posz_hbm.at[pidx_v.at[j, a]], accx, psx, add=True)

        def drain_write(j, accx, psx):
            for a in range(align):
                pltpu.make_async_copy(zero_hbm, accx, psx).wait()
            pltpu.sync_copy(accx, p_hbm.at[pl.ds(pbase + j * _CH, _CH)])

        def pbody(g2, c):
            j = 2 * g2
            fire(j, acca, psema, zsema)
            fire(j + 1, accb, psemb, zsemb)
            drain_write(j, acca, psema)
            pltpu.async_copy(zero_hbm, acca, zsema)
            drain_write(j + 1, accb, psemb)
            pltpu.async_copy(zero_hbm, accb, zsemb)
            return c

        lax.fori_loop(0, n_pchunks // 2, pbody, 0)
        if n_pchunks % 2:
            j = n_pchunks - 1
            fire(j, acca, psema, zsema)
            drain_write(j, acca, psema)

    return k(ids3, itax, word_emb, pos_z, jnp.zeros((_CH, h), jnp.float32))


def _tc_body(g_ref, vis_ref, ttt_ref, vtt_ref, rms_ref, p_ref, pos_ref,
             aux_ref, w_ref, out_ref):
    l, grp, h = g_ref.shape
    v = vis_ref.shape[0]
    d = vis_ref.shape[2]

    type2 = aux_ref[0:2, :]
    pev0 = aux_ref[2:3, :]
    pb = aux_ref[3:4, :]
    gam = aux_ref[4:5, :][:, None, :]
    bet = aux_ref[5:6, :][:, None, :]

    def ln3(x):
        m = jnp.mean(x, axis=-1, keepdims=True)
        xc = x - m
        var = jnp.mean(xc * xc, axis=-1, keepdims=True)
        return xc * lax.rsqrt(var + 1e-12) * gam + bet

    dn = (((0,), (0,)), ((), ()))
    ttype = lax.dot_general(ttt_ref[0], type2, dn,
                            preferred_element_type=jnp.float32)
    text = g_ref[...] + pos_ref[...][:, None, :] + ttype.reshape(l, grp, h)

    vf = jnp.dot(vis_ref[...].reshape(v * grp, d).astype(jnp.bfloat16),
                 w_ref[...], preferred_element_type=jnp.float32)
    vtype = lax.dot_general(vtt_ref[0], type2, dn,
                            preferred_element_type=jnp.float32)
    rowi = lax.broadcasted_iota(jnp.int32, (v * grp, v * grp), 0)
    colj = lax.broadcasted_iota(jnp.int32, (v * grp, v * grp), 1)
    sel = jnp.where(colj == rowi,
                    jnp.broadcast_to(rms_ref[0], (v * grp, v * grp)), 0.0)
    pev = jnp.dot(sel, p_ref[...].reshape(v * grp, h),
                  preferred_element_type=jnp.float32)
    vis = vf + pev + vtype + (pb + pev0)

    out_ref[0:l] = ln3(text)
    out_ref[l:l + v] = ln3(vis.reshape(v, grp, h))


def kernel(input_ids, token_type_ids, visual_feats, visual_feature_type_ids,
           image_text_alignment, word_emb, pos_emb, type_emb, pos_emb_visual,
           proj_W, proj_b, ln_gamma, ln_beta):
    b, l = input_ids.shape
    v = visual_feats.shape[1]
    h = word_emb.shape[1]
    align = image_text_alignment.shape[2]

    ids3 = (input_ids.astype(jnp.int32).T
            .reshape(_NW, b * l // (_NW * _CH), _CH))
    itax = (image_text_alignment.astype(jnp.int32).transpose(1, 0, 2)
            .reshape(_NW, b * v // (_NW * _CH), _CH, align)
            .transpose(0, 1, 3, 2))
    pos_z = pos_emb.at[0].set(0.0)

    g, p = _sc_gather(ids3, itax, word_emb, pos_z)
    g = g.reshape(l, b, h)
    p = p.reshape(v, b, h)
    vis_t = visual_feats.transpose(1, 0, 2)

    grp = 16
    ng = b // grp
    two = jnp.arange(2, dtype=token_type_ids.dtype)
    ttt = (token_type_ids.T.reshape(l, ng, grp)[None]
           == two[:, None, None, None]).astype(jnp.float32)
    ttt = ttt.transpose(2, 0, 1, 3).reshape(ng, 2, l * grp)
    vtt = (visual_feature_type_ids.T.reshape(v, ng, grp)[None]
           == two[:, None, None, None]).astype(jnp.float32)
    vtt = vtt.transpose(2, 0, 1, 3).reshape(ng, 2, v * grp)
    msum = (image_text_alignment != 0).sum(axis=2).astype(jnp.float32)
    rms = (1.0 / jnp.maximum(msum, 1.0)).T.reshape(v, ng, grp)
    rms = rms.transpose(1, 0, 2).reshape(ng, 1, v * grp)

    aux = jnp.zeros((8, h), jnp.float32)
    aux = aux.at[0].set(type_emb[0]).at[1].set(type_emb[1])
    aux = aux.at[2].set(pos_emb_visual[0]).at[3].set(proj_b)
    aux = aux.at[4].set(ln_gamma).at[5].set(ln_beta)

    pos_l = pos_emb[:l]
    d = visual_feats.shape[2]

    out = pl.pallas_call(
        _tc_body,
        grid=(ng,),
        in_specs=[
            pl.BlockSpec((l, grp, h), lambda i: (0, i, 0)),
            pl.BlockSpec((v, grp, d), lambda i: (0, i, 0)),
            pl.BlockSpec((1, 2, grp * l), lambda i: (i, 0, 0)),
            pl.BlockSpec((1, 2, grp * v), lambda i: (i, 0, 0)),
            pl.BlockSpec((1, 1, grp * v), lambda i: (i, 0, 0)),
            pl.BlockSpec((v, grp, h), lambda i: (0, i, 0)),
            pl.BlockSpec((l, h), lambda i: (0, 0)),
            pl.BlockSpec((8, h), lambda i: (0, 0)),
            pl.BlockSpec((d, h), lambda i: (0, 0)),
        ],
        out_specs=pl.BlockSpec((l + v, grp, h), lambda i: (0, i, 0)),
        out_shape=jax.ShapeDtypeStruct((l + v, b, h), jnp.float32),
        compiler_params=pltpu.CompilerParams(
            dimension_semantics=("arbitrary",)),
    )(g, vis_t, ttt, vtt, rms, p, pos_l, aux,
      proj_W.astype(jnp.bfloat16))
    return jnp.transpose(out, (1, 0, 2))


# speedup vs baseline: 1.3246x; 1.0521x over previous
import functools

import jax
import jax.numpy as jnp
from jax import lax
from jax.experimental import pallas as pl
from jax.experimental.pallas import tpu as pltpu
from jax.experimental.pallas import tpu_sc as plsc

_NC, _NS = 2, 16
_NW = _NC * _NS
_CH = 128


def _sc_gather(ids3, itax, word_emb, pos_z):
    nw, n_chunks, _ = ids3.shape
    _, n_pchunks, align, _ = itax.shape
    bl = nw * n_chunks * _CH
    bp = nw * n_pchunks * _CH
    _, h = word_emb.shape
    rows_w = bl // _NW
    prow_w = bp // _NW

    mesh = plsc.VectorSubcoreMesh(
        core_axis_name="c", subcore_axis_name="s",
        num_cores=_NC, num_subcores=_NS)

    @functools.partial(
        pl.kernel,
        out_type=(jax.ShapeDtypeStruct((bl, h), jnp.float32),
                  jax.ShapeDtypeStruct((bp, h), jnp.float32)),
        mesh=mesh,
        scratch_types=[
            pltpu.VMEM((n_chunks, _CH), jnp.int32),
            pltpu.VMEM((n_pchunks, align, _CH), jnp.int32),
            pltpu.VMEM((_CH, h), jnp.float32),
            pltpu.VMEM((_CH, h), jnp.float32),
            pltpu.VMEM((_CH, h), jnp.float32),
            pltpu.VMEM((_CH, h), jnp.float32),
            pltpu.SemaphoreType.DMA,
            pltpu.SemaphoreType.DMA,
            pltpu.SemaphoreType.DMA,
            pltpu.SemaphoreType.DMA,
            pltpu.SemaphoreType.DMA,
            pltpu.SemaphoreType.DMA,
        ],
    )
    def k(ids_hbm, ita_hbm, word_hbm, posz_hbm, zero_hbm, g_hbm, p_hbm,
          idx_v, pidx_v, bufa, bufb, acca, accb,
          sema, semb, psema, psemb, zsema, zsemb):
        wid = lax.axis_index("s") * _NC + lax.axis_index("c")
        gbase = wid * rows_w
        pbase = wid * prow_w
        pltpu.sync_copy(ids_hbm.at[wid], idx_v)
        pltpu.sync_copy(ita_hbm.at[wid], pidx_v)

        pltpu.async_copy(word_hbm.at[idx_v.at[0]], bufa, sema)

        def tbody(g2, c):
            j = 2 * g2
            pltpu.async_copy(word_hbm.at[idx_v.at[j + 1]], bufb, semb)
            pltpu.make_async_copy(word_hbm.at[idx_v.at[j]], bufa, sema).wait()
            pltpu.sync_copy(bufa, g_hbm.at[pl.ds(gbase + j * _CH, _CH)])

            @pl.when(j + 2 < n_chunks)
            def _():
                pltpu.async_copy(word_hbm.at[idx_v.at[j + 2]], bufa, sema)

            pltpu.make_async_copy(
                word_hbm.at[idx_v.at[j + 1]], bufb, semb).wait()
            pltpu.sync_copy(bufb, g_hbm.at[pl.ds(gbase + (j + 1) * _CH, _CH)])
            return c

        lax.fori_loop(0, n_chunks // 2, tbody, 0)

        pltpu.async_copy(zero_hbm, acca, zsema)
        pltpu.async_copy(zero_hbm, accb, zsemb)

        def fire(j, accx, psx, zsx):
            pltpu.make_async_copy(zero_hbm, accx, zsx).wait()
            for a in range(align):
                pltpu.async_copy(
                    posz_hbm.at[pidx_v.at[j, a]], accx, psx, add=True)

        def drain_write(j, accx, psx):
            for a in range(align):
                pltpu.make_async_copy(zero_hbm, accx, psx).wait()
            pltpu.sync_copy(accx, p_hbm.at[pl.ds(pbase + j * _CH, _CH)])

        def pbody(g2, c):
            j = 2 * g2
            fire(j, acca, psema, zsema)
            fire(j + 1, accb, psemb, zsemb)
            drain_write(j, acca, psema)
            pltpu.async_copy(zero_hbm, acca, zsema)
            drain_write(j + 1, accb, psemb)
            pltpu.async_copy(zero_hbm, accb, zsemb)
            return c

        lax.fori_loop(0, n_pchunks // 2, pbody, 0)
        if n_pchunks % 2:
            j = n_pchunks - 1
            fire(j, acca, psema, zsema)
            drain_write(j, acca, psema)

    return k(ids3, itax, word_emb, pos_z, jnp.zeros((_CH, h), jnp.float32))


def _tc_body(g_ref, vis_ref, ttt_ref, vtt_ref, rms_ref, p_ref, pos_ref,
             aux_ref, w_ref, out_ref):
    l, grp, h = g_ref.shape
    v = vis_ref.shape[0]
    d = vis_ref.shape[2]

    type2 = aux_ref[0:2, :]
    pev0 = aux_ref[2:3, :]
    pb = aux_ref[3:4, :]
    gam = aux_ref[4:5, :][:, None, :]
    bet = aux_ref[5:6, :][:, None, :]

    def ln3(x):
        m = jnp.mean(x, axis=-1, keepdims=True)
        xc = x - m
        var = jnp.mean(xc * xc, axis=-1, keepdims=True)
        return xc * lax.rsqrt(var + 1e-12) * gam + bet

    dn = (((0,), (0,)), ((), ()))
    ttype = lax.dot_general(ttt_ref[0], type2, dn,
                            preferred_element_type=jnp.float32)
    text = g_ref[...] + pos_ref[...][:, None, :] + ttype.reshape(l, grp, h)

    vf = jnp.dot(vis_ref[...].reshape(v * grp, d).astype(jnp.bfloat16),
                 w_ref[...], preferred_element_type=jnp.float32)
    vtype = lax.dot_general(vtt_ref[0], type2, dn,
                            preferred_element_type=jnp.float32)
    rowi = lax.broadcasted_iota(jnp.int32, (v * grp, v * grp), 0)
    colj = lax.broadcasted_iota(jnp.int32, (v * grp, v * grp), 1)
    sel = jnp.where(colj == rowi,
                    jnp.broadcast_to(rms_ref[0], (v * grp, v * grp)), 0.0)
    pev = jnp.dot(sel, p_ref[...].reshape(v * grp, h),
                  preferred_element_type=jnp.float32)
    vis = vf + pev + vtype + (pb + pev0)

    out_ref[0:l] = ln3(text)
    out_ref[l:l + v] = ln3(vis.reshape(v, grp, h))


def kernel(input_ids, token_type_ids, visual_feats, visual_feature_type_ids,
           image_text_alignment, word_emb, pos_emb, type_emb, pos_emb_visual,
           proj_W, proj_b, ln_gamma, ln_beta):
    b, l = input_ids.shape
    v = visual_feats.shape[1]
    h = word_emb.shape[1]
    align = image_text_alignment.shape[2]

    ids3 = (input_ids.astype(jnp.int32).T
            .reshape(_NW, b * l // (_NW * _CH), _CH))
    itax = (image_text_alignment.astype(jnp.int32).transpose(1, 0, 2)
            .reshape(_NW, b * v // (_NW * _CH), _CH, align)
            .transpose(0, 1, 3, 2))
    pos_z = pos_emb.at[0].set(0.0)

    g, p = _sc_gather(ids3, itax, word_emb, pos_z)
    g = g.reshape(l, b, h)
    p = p.reshape(v, b, h)
    vis_t = visual_feats.transpose(1, 0, 2)

    grp = 32
    ng = b // grp
    two = jnp.arange(2, dtype=token_type_ids.dtype)
    ttt = (token_type_ids.T.reshape(l, ng, grp)[None]
           == two[:, None, None, None]).astype(jnp.float32)
    ttt = ttt.transpose(2, 0, 1, 3).reshape(ng, 2, l * grp)
    vtt = (visual_feature_type_ids.T.reshape(v, ng, grp)[None]
           == two[:, None, None, None]).astype(jnp.float32)
    vtt = vtt.transpose(2, 0, 1, 3).reshape(ng, 2, v * grp)
    msum = (image_text_alignment != 0).sum(axis=2).astype(jnp.float32)
    rms = (1.0 / jnp.maximum(msum, 1.0)).T.reshape(v, ng, grp)
    rms = rms.transpose(1, 0, 2).reshape(ng, 1, v * grp)

    aux = jnp.zeros((8, h), jnp.float32)
    aux = aux.at[0].set(type_emb[0]).at[1].set(type_emb[1])
    aux = aux.at[2].set(pos_emb_visual[0]).at[3].set(proj_b)
    aux = aux.at[4].set(ln_gamma).at[5].set(ln_beta)

    pos_l = pos_emb[:l]
    d = visual_feats.shape[2]

    out = pl.pallas_call(
        _tc_body,
        grid=(ng,),
        in_specs=[
            pl.BlockSpec((l, grp, h), lambda i: (0, i, 0)),
            pl.BlockSpec((v, grp, d), lambda i: (0, i, 0)),
            pl.BlockSpec((1, 2, grp * l), lambda i: (i, 0, 0)),
            pl.BlockSpec((1, 2, grp * v), lambda i: (i, 0, 0)),
            pl.BlockSpec((1, 1, grp * v), lambda i: (i, 0, 0)),
            pl.BlockSpec((v, grp, h), lambda i: (0, i, 0)),
            pl.BlockSpec((l, h), lambda i: (0, 0)),
            pl.BlockSpec((8, h), lambda i: (0, 0)),
            pl.BlockSpec((d, h), lambda i: (0, 0)),
        ],
        out_specs=pl.BlockSpec((l + v, grp, h), lambda i: (0, i, 0)),
        out_shape=jax.ShapeDtypeStruct((l + v, b, h), jnp.float32),
        compiler_params=pltpu.CompilerParams(
            dimension_semantics=("arbitrary",)),
    )(g, vis_t, ttt, vtt, rms, p, pos_l, aux,
      proj_W.astype(jnp.bfloat16))
    return jnp.transpose(out, (1, 0, 2))
